# baseline (speedup 1.0000x reference)
import functools

import jax
import jax.numpy as jnp
from jax import lax
from jax.experimental import pallas as pl
from jax.experimental.pallas import tpu as pltpu
from jax.experimental.pallas import tpu_sc as plsc

B = 16384
D = 32
K = 5
NC = 2
NS = 16
NW = NC * NS
BPW = B // NW
CH = 128
NCH = BPW // CH
NEG_PER_W = BPW * K
NEG_NCH = NEG_PER_W // CH
NDOT = BPW * (K + 1)


def _sc_body(vi_idx_h, vj_idx_h, neg_idx_h, nodes_h, ctx_h, out_h,
             vi_idx, vj_idx, neg_idx, vi_rows, vj_rows, neg_rows,
             dots, acc_v, sem):
    wid = lax.axis_index("s") * NC + lax.axis_index("c")

    pltpu.sync_copy(vi_idx_h.at[wid], vi_idx)
    pltpu.sync_copy(vj_idx_h.at[wid], vj_idx)
    pltpu.sync_copy(neg_idx_h.at[wid], neg_idx)

    copies = []
    for c in range(NCH):
        copies.append(pltpu.async_copy(
            nodes_h.at[vi_idx.at[c]], vi_rows.at[pl.ds(c * CH, CH)], sem))
        copies.append(pltpu.async_copy(
            ctx_h.at[vj_idx.at[c]], vj_rows.at[pl.ds(c * CH, CH)], sem))
    for c in range(NEG_NCH):
        copies.append(pltpu.async_copy(
            ctx_h.at[neg_idx.at[c]], neg_rows.at[pl.ds(c * CH, CH)], sem))
    for cp in copies:
        cp.wait()

    last = lax.iota(jnp.int32, 16) == 15

    def bstep(b, ivec):
        vi0 = vi_rows[b, pl.ds(0, 16)]
        vi1 = vi_rows[b, pl.ds(16, 16)]
        vj0 = vj_rows[b, pl.ds(0, 16)]
        vj1 = vj_rows[b, pl.ds(16, 16)]
        cpos = plsc.cumsum(vi0 * vj0 + vi1 * vj1)
        plsc.store_scatter(dots, [ivec], cpos, mask=last)
        nvi0 = -vi0
        nvi1 = -vi1
        for k in range(K):
            n0 = neg_rows[b * K + k, pl.ds(0, 16)]
            n1 = neg_rows[b * K + k, pl.ds(16, 16)]
            cneg = plsc.cumsum(nvi0 * n0 + nvi1 * n1)
            plsc.store_scatter(dots, [ivec + (k + 1)], cneg, mask=last)
        return ivec + (K + 1)

    lax.fori_loop(0, BPW, bstep, jnp.zeros((16,), jnp.int32))

    def sstep(j, a):
        dv = dots[pl.ds(j * 16, 16)]
        return a + 1.0 / (1.0 + jnp.exp(-dv))

    total = lax.fori_loop(0, NDOT // 16, sstep, jnp.zeros((16,), jnp.float32))
    acc_v[...] = total
    pltpu.sync_copy(acc_v, out_h.at[wid])


_sc_call = functools.partial(
    pl.kernel,
    mesh=plsc.VectorSubcoreMesh(core_axis_name="c", subcore_axis_name="s"),
    out_type=jax.ShapeDtypeStruct((NW, 16), jnp.float32),
    compiler_params=pltpu.CompilerParams(
        needs_layout_passes=False, use_tc_tiling_on_sc=False),
    scratch_types=[
        pltpu.VMEM((NCH, CH), jnp.int32),
        pltpu.VMEM((NCH, CH), jnp.int32),
        pltpu.VMEM((NEG_NCH, CH), jnp.int32),
        pltpu.VMEM((BPW, D), jnp.float32),
        pltpu.VMEM((BPW, D), jnp.float32),
        pltpu.VMEM((NEG_PER_W, D), jnp.float32),
        pltpu.VMEM((NDOT,), jnp.float32),
        pltpu.VMEM((16,), jnp.float32),
        pltpu.SemaphoreType.DMA,
    ],
)(_sc_body)


def kernel(v_i, v_j, negsamples, nodes_embeddings, contextnodes_embeddings):
    vi3 = v_i.astype(jnp.int32).reshape(NW, NCH, CH)
    vj3 = v_j.astype(jnp.int32).reshape(NW, NCH, CH)
    neg3 = negsamples.astype(jnp.int32).reshape(NW, NEG_NCH, CH)
    partials = _sc_call(vi3, vj3, neg3, nodes_embeddings,
                        contextnodes_embeddings)
    return -(jnp.sum(partials) / B)
